# Initial kernel scaffold; baseline (speedup 1.0000x reference)
import jax
import jax.numpy as jnp
from jax.experimental import pallas as pl


def kernel(X, edge_index, cent_n_id, Wl, bl, Wr, br, node_emb, W_ih, W_hh, b_ih, b_hh, W_fc, b_fc):
    raise NotImplementedError("write your pallas kernel here")


# speedup vs baseline: 10.4855x; 10.4855x over previous
import functools

import jax
import jax.numpy as jnp
from jax import lax
from jax.experimental import pallas as pl
from jax.experimental.pallas import tpu as pltpu
from jax.experimental.pallas import tpu_sc as plsc

_NC = 2
_NS = 16
_NW = _NC * _NS

_CH = 125
_WB = 160


def _proj_body(T, H, x_ref, wl_ref, wr_ref, z_ref, r_ref):
    x = x_ref[0]
    bn = x.shape[0]
    xf = x.reshape(bn * T, x.shape[2])
    z = jnp.dot(xf, wl_ref[...], preferred_element_type=jnp.float32)
    r = jnp.dot(xf, wr_ref[...], preferred_element_type=jnp.float32)
    z_ref[...] = z.reshape(bn, T, H).transpose(1, 0, 2)
    r_ref[...] = r.reshape(bn, T, H).transpose(1, 0, 2)


def _make_proj(B, N, T, F, H, bn):
    return pl.pallas_call(
        functools.partial(_proj_body, T, H),
        grid=(B, N // bn),
        in_specs=[
            pl.BlockSpec((1, bn, T, F), lambda b, j: (b, j, 0, 0)),
            pl.BlockSpec((F, H), lambda b, j: (0, 0)),
            pl.BlockSpec((F, H), lambda b, j: (0, 0)),
        ],
        out_specs=[
            pl.BlockSpec((T, bn, H), lambda b, j: (b, j, 0)),
            pl.BlockSpec((T, bn, H), lambda b, j: (b, j, 0)),
        ],
        out_shape=[
            jax.ShapeDtypeStruct((B * T, N, H), jnp.float32),
            jax.ShapeDtypeStruct((B * T, N, H), jnp.float32),
        ],
    )


def _make_sc_segsum(T24, NPAD, H, NCH):
    NPW = NPAD // _NS
    NWBC = NPW // _WB

    def body(z_hbm, src_hbm, dst_hbm, zeros_hbm, zeros8_hbm, ones8_hbm,
             spart_hbm, degp_hbm,
             src_v, dst_v, gbuf, zbuf, wb, obuf, z8, acc_sh, dacc_sh, sem):
        c = lax.axis_index("c")
        s = lax.axis_index("s")
        w = s * _NC + c
        row0 = s * NPW

        pltpu.sync_copy(src_hbm.at[w], src_v)
        pltpu.sync_copy(dst_hbm.at[w], dst_v)
        pltpu.sync_copy(zeros_hbm, zbuf)
        pltpu.sync_copy(zeros8_hbm, z8)
        pltpu.sync_copy(ones8_hbm, obuf)

        pltpu.sync_copy(z8, dacc_sh.at[pl.ds(row0, NPW)])
        plsc.subcore_barrier()

        @pl.loop(0, NCH)
        def _deg(k):
            pltpu.sync_copy(obuf, dacc_sh.at[dst_v.at[k]], add=True)

        plsc.subcore_barrier()
        pltpu.sync_copy(dacc_sh.at[pl.ds(row0, NPW)], z8)
        pltpu.sync_copy(z8, degp_hbm.at[c, pl.ds(row0, NPW)])

        @pl.loop(0, T24)
        def _t(t):
            @pl.loop(0, NWBC)
            def _z(i):
                pltpu.sync_copy(zbuf, acc_sh.at[pl.ds(row0 + i * _WB, _WB)])

            plsc.subcore_barrier()
            zt = z_hbm.at[t]

            @pl.loop(0, NCH)
            def _k(k):
                pltpu.async_copy(zt.at[src_v.at[k]], gbuf, sem).wait()
                pltpu.sync_copy(gbuf, acc_sh.at[dst_v.at[k]], add=True)

            plsc.subcore_barrier()

            @pl.loop(0, NWBC)
            def _wbk(i):
                pltpu.sync_copy(acc_sh.at[pl.ds(row0 + i * _WB, _WB)], wb)
                pltpu.sync_copy(
                    wb, spart_hbm.at[c, t, pl.ds(row0 + i * _WB, _WB)])

    mesh = plsc.VectorSubcoreMesh(
        core_axis_name="c", subcore_axis_name="s",
        num_cores=_NC, num_subcores=_NS)
    return pl.kernel(
        body,
        out_type=[
            jax.ShapeDtypeStruct((_NC, T24, NPAD, H), jnp.float32),
            jax.ShapeDtypeStruct((_NC, NPAD, 8), jnp.float32),
        ],
        mesh=mesh,
        scratch_types=[
            pltpu.VMEM((NCH, _CH), jnp.int32),
            pltpu.VMEM((NCH, _CH), jnp.int32),
            pltpu.VMEM((_CH, H), jnp.float32),
            pltpu.VMEM((_WB, H), jnp.float32),
            pltpu.VMEM((_WB, H), jnp.float32),
            pltpu.VMEM((_CH, 8), jnp.float32),
            pltpu.VMEM((NPW, 8), jnp.float32),
            pltpu.VMEM_SHARED((NPAD, H), jnp.float32),
            pltpu.VMEM_SHARED((NPAD, 8), jnp.float32),
            pltpu.SemaphoreType.DMA,
        ],
        compiler_params=pltpu.CompilerParams(use_tc_tiling_on_sc=False),
    )


def _tail_body(B, T, H, r_ref, sp_ref, dg_ref, emb_ref, bl_ref, br_ref,
               wih_ref, whh_ref, bih_ref, bhh_ref, wfc_ref, bfc_ref, out_ref):
    d = dg_ref[0, :, 0:1] + dg_ref[1, :, 0:1]
    rinv = 1.0 / jnp.maximum(d, 1.0)
    emb = emb_ref[...]
    bl = bl_ref[...]
    br = br_ref[...]
    wih = wih_ref[...]
    whh = whh_ref[...]
    bih = bih_ref[...]
    bhh = bhh_ref[...]
    bn = emb.shape[0]
    for b in range(B):
        h = jnp.zeros((bn, H), jnp.float32)
        for t in range(T):
            bt = b * T + t
            sp = sp_ref[0, bt] + sp_ref[1, bt]
            out1 = jax.nn.relu(r_ref[bt] + br + sp * rinv + bl)
            inp = out1 + emb
            gx = jnp.dot(inp, wih, preferred_element_type=jnp.float32) + bih
            gh = jnp.dot(h, whh, preferred_element_type=jnp.float32) + bhh
            r_g = jax.nn.sigmoid(gx[:, :H] + gh[:, :H])
            z_g = jax.nn.sigmoid(gx[:, H:2 * H] + gh[:, H:2 * H])
            n_g = jnp.tanh(gx[:, 2 * H:] + r_g * gh[:, 2 * H:])
            h = (1.0 - z_g) * n_g + z_g * h
        out_ref[b] = (jnp.dot(h, wfc_ref[...], preferred_element_type=jnp.float32)
                      + bfc_ref[...])


def _make_tail(B, N, T, H, TOUT, bn):
    return pl.pallas_call(
        functools.partial(_tail_body, B, T, H),
        grid=(N // bn,),
        in_specs=[
            pl.BlockSpec((B * T, bn, H), lambda j: (0, j, 0)),
            pl.BlockSpec((_NC, B * T, bn, H), lambda j: (0, 0, j, 0)),
            pl.BlockSpec((_NC, bn, 8), lambda j: (0, j, 0)),
            pl.BlockSpec((bn, H), lambda j: (j, 0)),
            pl.BlockSpec((1, H), lambda j: (0, 0)),
            pl.BlockSpec((1, H), lambda j: (0, 0)),
            pl.BlockSpec((H, 3 * H), lambda j: (0, 0)),
            pl.BlockSpec((H, 3 * H), lambda j: (0, 0)),
            pl.BlockSpec((1, 3 * H), lambda j: (0, 0)),
            pl.BlockSpec((1, 3 * H), lambda j: (0, 0)),
            pl.BlockSpec((H, TOUT), lambda j: (0, 0)),
            pl.BlockSpec((1, TOUT), lambda j: (0, 0)),
        ],
        out_specs=pl.BlockSpec((B, bn, TOUT), lambda j: (0, j, 0)),
        out_shape=jax.ShapeDtypeStruct((B, N, TOUT), jnp.float32),
    )


def kernel(X, edge_index, cent_n_id, Wl, bl, Wr, br, node_emb,
           W_ih, W_hh, b_ih, b_hh, W_fc, b_fc):
    B, N, T, F = X.shape
    H = Wl.shape[1]
    TOUT = W_fc.shape[1]
    E = edge_index.shape[1]
    T24 = B * T
    EPW = E // _NW
    NCH = EPW // _CH
    NPAD = -(-N // (_NS * _WB)) * (_NS * _WB)
    NPW = NPAD // _NS

    z, r = _make_proj(B, N, T, F, H, bn=400)(X, Wl, Wr)

    src_r = edge_index[0].reshape(_NW, NCH, _CH)
    dst_r = edge_index[1].reshape(_NW, NCH, _CH)
    zeros = jnp.zeros((_WB, H), jnp.float32)
    zeros8 = jnp.zeros((NPW, 8), jnp.float32)
    ones8 = jnp.ones((_CH, 8), jnp.float32)
    spart, degp = _make_sc_segsum(T24, NPAD, H, NCH)(
        z, src_r, dst_r, zeros, zeros8, ones8)

    emb = node_emb[cent_n_id]
    out = _make_tail(B, N, T, H, TOUT, bn=400)(
        r, spart, degp, emb,
        bl.reshape(1, H), br.reshape(1, H), W_ih, W_hh,
        b_ih.reshape(1, 3 * H), b_hh.reshape(1, 3 * H),
        W_fc, b_fc.reshape(1, TOUT))
    return out


# speedup vs baseline: 14.4008x; 1.3734x over previous
import functools

import jax
import jax.numpy as jnp
from jax import lax
from jax.experimental import pallas as pl
from jax.experimental.pallas import tpu as pltpu
from jax.experimental.pallas import tpu_sc as plsc

_NC = 2
_NS = 16
_NW = _NC * _NS

_CH = 125
_WB = 160


def _proj_body(T, H, x_ref, wl_ref, wr_ref, z_ref, r_ref):
    x = x_ref[0]
    bn = x.shape[0]
    xf = x.reshape(bn * T, x.shape[2])
    z = jnp.dot(xf, wl_ref[...], preferred_element_type=jnp.float32)
    r = jnp.dot(xf, wr_ref[...], preferred_element_type=jnp.float32)
    z_ref[...] = z.reshape(bn, T, H).transpose(1, 0, 2)
    r_ref[...] = r.reshape(bn, T, H).transpose(1, 0, 2)


def _make_proj(B, N, T, F, H, bn):
    return pl.pallas_call(
        functools.partial(_proj_body, T, H),
        grid=(B, N // bn),
        in_specs=[
            pl.BlockSpec((1, bn, T, F), lambda b, j: (b, j, 0, 0)),
            pl.BlockSpec((F, H), lambda b, j: (0, 0)),
            pl.BlockSpec((F, H), lambda b, j: (0, 0)),
        ],
        out_specs=[
            pl.BlockSpec((T, bn, H), lambda b, j: (b, j, 0)),
            pl.BlockSpec((T, bn, H), lambda b, j: (b, j, 0)),
        ],
        out_shape=[
            jax.ShapeDtypeStruct((B * T, N, H), jnp.float32),
            jax.ShapeDtypeStruct((B * T, N, H), jnp.float32),
        ],
    )


def _make_sc_segsum(T24, NPAD, H, NCH):
    NPW = NPAD // _NS
    NWBC = NPW // _WB

    def body(z_hbm, src_hbm, dst_hbm, zeros_hbm, zeros8_hbm, ones8_hbm,
             spart_hbm, degp_hbm,
             src_v, dst_v, gbuf, gbuf2, zbuf, wb, obuf, z8, acc_sh, dacc_sh,
             sem, sem2):
        c = lax.axis_index("c")
        s = lax.axis_index("s")
        w = s * _NC + c
        row0 = s * NPW

        pltpu.sync_copy(src_hbm.at[w], src_v)
        pltpu.sync_copy(dst_hbm.at[w], dst_v)
        pltpu.sync_copy(zeros_hbm, zbuf)
        pltpu.sync_copy(zeros8_hbm, z8)
        pltpu.sync_copy(ones8_hbm, obuf)

        pltpu.sync_copy(z8, dacc_sh.at[pl.ds(row0, NPW)])
        plsc.subcore_barrier()

        @pl.loop(0, NCH)
        def _deg(k):
            pltpu.sync_copy(obuf, dacc_sh.at[dst_v.at[k]], add=True)

        plsc.subcore_barrier()
        pltpu.sync_copy(dacc_sh.at[pl.ds(row0, NPW)], z8)
        pltpu.sync_copy(z8, degp_hbm.at[c, pl.ds(row0, NPW)])

        @pl.loop(0, T24)
        def _t(t):
            @pl.loop(0, NWBC)
            def _z(i):
                pltpu.sync_copy(zbuf, acc_sh.at[pl.ds(row0 + i * _WB, _WB)])

            plsc.subcore_barrier()
            zt = z_hbm.at[t]

            pltpu.async_copy(zt.at[src_v.at[0]], gbuf, sem)

            @pl.loop(0, NCH - 2, step=2)
            def _k(g):
                dB = pltpu.async_copy(zt.at[src_v.at[g + 1]], gbuf2, sem2)
                pltpu.make_async_copy(zt.at[src_v.at[g]], gbuf, sem).wait()
                pltpu.sync_copy(gbuf, acc_sh.at[dst_v.at[g]], add=True)
                pltpu.async_copy(zt.at[src_v.at[g + 2]], gbuf, sem)
                dB.wait()
                pltpu.sync_copy(gbuf2, acc_sh.at[dst_v.at[g + 1]], add=True)

            dB = pltpu.async_copy(zt.at[src_v.at[NCH - 1]], gbuf2, sem2)
            pltpu.make_async_copy(zt.at[src_v.at[NCH - 2]], gbuf, sem).wait()
            pltpu.sync_copy(gbuf, acc_sh.at[dst_v.at[NCH - 2]], add=True)
            dB.wait()
            pltpu.sync_copy(gbuf2, acc_sh.at[dst_v.at[NCH - 1]], add=True)

            plsc.subcore_barrier()

            @pl.loop(0, NWBC)
            def _wbk(i):
                pltpu.sync_copy(acc_sh.at[pl.ds(row0 + i * _WB, _WB)], wb)
                pltpu.sync_copy(
                    wb, spart_hbm.at[c, t, pl.ds(row0 + i * _WB, _WB)])

    mesh = plsc.VectorSubcoreMesh(
        core_axis_name="c", subcore_axis_name="s",
        num_cores=_NC, num_subcores=_NS)
    return pl.kernel(
        body,
        out_type=[
            jax.ShapeDtypeStruct((_NC, T24, NPAD, H), jnp.float32),
            jax.ShapeDtypeStruct((_NC, NPAD, 8), jnp.float32),
        ],
        mesh=mesh,
        scratch_types=[
            pltpu.VMEM((NCH, _CH), jnp.int32),
            pltpu.VMEM((NCH, _CH), jnp.int32),
            pltpu.VMEM((_CH, H), jnp.float32),
            pltpu.VMEM((_CH, H), jnp.float32),
            pltpu.VMEM((_WB, H), jnp.float32),
            pltpu.VMEM((_WB, H), jnp.float32),
            pltpu.VMEM((_CH, 8), jnp.float32),
            pltpu.VMEM((NPW, 8), jnp.float32),
            pltpu.VMEM_SHARED((NPAD, H), jnp.float32),
            pltpu.VMEM_SHARED((NPAD, 8), jnp.float32),
            pltpu.SemaphoreType.DMA,
            pltpu.SemaphoreType.DMA,
        ],
        compiler_params=pltpu.CompilerParams(use_tc_tiling_on_sc=False),
    )


def _tail_body(B, T, H, r_ref, sp_ref, dg_ref, emb_ref, bl_ref, br_ref,
               wih_ref, whh_ref, bih_ref, bhh_ref, wfc_ref, bfc_ref, out_ref):
    d = dg_ref[0, :, 0:1] + dg_ref[1, :, 0:1]
    rinv = 1.0 / jnp.maximum(d, 1.0)
    emb = emb_ref[...]
    bl = bl_ref[...]
    br = br_ref[...]
    wih = wih_ref[...]
    whh = whh_ref[...]
    bih = bih_ref[...]
    bhh = bhh_ref[...]
    bn = emb.shape[0]
    for b in range(B):
        h = jnp.zeros((bn, H), jnp.float32)
        for t in range(T):
            bt = b * T + t
            sp = sp_ref[0, bt] + sp_ref[1, bt]
            out1 = jax.nn.relu(r_ref[bt] + br + sp * rinv + bl)
            inp = out1 + emb
            gx = jnp.dot(inp, wih, preferred_element_type=jnp.float32) + bih
            gh = jnp.dot(h, whh, preferred_element_type=jnp.float32) + bhh
            r_g = jax.nn.sigmoid(gx[:, :H] + gh[:, :H])
            z_g = jax.nn.sigmoid(gx[:, H:2 * H] + gh[:, H:2 * H])
            n_g = jnp.tanh(gx[:, 2 * H:] + r_g * gh[:, 2 * H:])
            h = (1.0 - z_g) * n_g + z_g * h
        out_ref[b] = (jnp.dot(h, wfc_ref[...], preferred_element_type=jnp.float32)
                      + bfc_ref[...])


def _make_tail(B, N, T, H, TOUT, bn):
    return pl.pallas_call(
        functools.partial(_tail_body, B, T, H),
        grid=(N // bn,),
        in_specs=[
            pl.BlockSpec((B * T, bn, H), lambda j: (0, j, 0)),
            pl.BlockSpec((_NC, B * T, bn, H), lambda j: (0, 0, j, 0)),
            pl.BlockSpec((_NC, bn, 8), lambda j: (0, j, 0)),
            pl.BlockSpec((bn, H), lambda j: (j, 0)),
            pl.BlockSpec((1, H), lambda j: (0, 0)),
            pl.BlockSpec((1, H), lambda j: (0, 0)),
            pl.BlockSpec((H, 3 * H), lambda j: (0, 0)),
            pl.BlockSpec((H, 3 * H), lambda j: (0, 0)),
            pl.BlockSpec((1, 3 * H), lambda j: (0, 0)),
            pl.BlockSpec((1, 3 * H), lambda j: (0, 0)),
            pl.BlockSpec((H, TOUT), lambda j: (0, 0)),
            pl.BlockSpec((1, TOUT), lambda j: (0, 0)),
        ],
        out_specs=pl.BlockSpec((B, bn, TOUT), lambda j: (0, j, 0)),
        out_shape=jax.ShapeDtypeStruct((B, N, TOUT), jnp.float32),
    )


def kernel(X, edge_index, cent_n_id, Wl, bl, Wr, br, node_emb,
           W_ih, W_hh, b_ih, b_hh, W_fc, b_fc):
    B, N, T, F = X.shape
    H = Wl.shape[1]
    TOUT = W_fc.shape[1]
    E = edge_index.shape[1]
    T24 = B * T
    EPW = E // _NW
    NCH = EPW // _CH
    NPAD = -(-N // (_NS * _WB)) * (_NS * _WB)
    NPW = NPAD // _NS

    z, r = _make_proj(B, N, T, F, H, bn=400)(X, Wl, Wr)

    src_r = edge_index[0].reshape(_NW, NCH, _CH)
    dst_r = edge_index[1].reshape(_NW, NCH, _CH)
    zeros = jnp.zeros((_WB, H), jnp.float32)
    zeros8 = jnp.zeros((NPW, 8), jnp.float32)
    ones8 = jnp.ones((_CH, 8), jnp.float32)
    spart, degp = _make_sc_segsum(T24, NPAD, H, NCH)(
        z, src_r, dst_r, zeros, zeros8, ones8)

    emb = node_emb[cent_n_id]
    out = _make_tail(B, N, T, H, TOUT, bn=400)(
        r, spart, degp, emb,
        bl.reshape(1, H), br.reshape(1, H), W_ih, W_hh,
        b_ih.reshape(1, 3 * H), b_hh.reshape(1, 3 * H),
        W_fc, b_fc.reshape(1, TOUT))
    return out


# speedup vs baseline: 14.8933x; 1.0342x over previous
import functools

import jax
import jax.numpy as jnp
from jax import lax
from jax.experimental import pallas as pl
from jax.experimental.pallas import tpu as pltpu
from jax.experimental.pallas import tpu_sc as plsc

_NC = 2
_NS = 16
_NW = _NC * _NS

_CH = 125
_WB = 160


def _proj_body(T, H, x_ref, wl_ref, wr_ref, z_ref, r_ref):
    x = x_ref[0]
    bn = x.shape[0]
    xf = x.reshape(bn * T, x.shape[2])
    z = jnp.dot(xf, wl_ref[...], preferred_element_type=jnp.float32)
    r = jnp.dot(xf, wr_ref[...], preferred_element_type=jnp.float32)
    z_ref[...] = z.reshape(bn, T, H).transpose(1, 0, 2)
    r_ref[...] = r.reshape(bn, T, H).transpose(1, 0, 2)


def _make_proj(B, N, T, F, H, bn):
    return pl.pallas_call(
        functools.partial(_proj_body, T, H),
        grid=(B, N // bn),
        in_specs=[
            pl.BlockSpec((1, bn, T, F), lambda b, j: (b, j, 0, 0)),
            pl.BlockSpec((F, H), lambda b, j: (0, 0)),
            pl.BlockSpec((F, H), lambda b, j: (0, 0)),
        ],
        out_specs=[
            pl.BlockSpec((T, bn, H), lambda b, j: (b, j, 0)),
            pl.BlockSpec((T, bn, H), lambda b, j: (b, j, 0)),
        ],
        out_shape=[
            jax.ShapeDtypeStruct((B * T, N, H), jnp.float32),
            jax.ShapeDtypeStruct((B * T, N, H), jnp.float32),
        ],
    )


def _make_sc_segsum(T24, NPAD, H, NCH):
    NPW = NPAD // _NS
    NWBC = NPW // _WB

    def body(z_hbm, src_hbm, dst_hbm, zeros_hbm, zeros8_hbm, ones8_hbm,
             spart_hbm, degp_hbm,
             src_v, dst_v, gbuf, gbuf2, zbuf, wb, obuf, z8, acc_sh, dacc_sh,
             sem, sem2):
        c = lax.axis_index("c")
        s = lax.axis_index("s")
        w = s * _NC + c
        row0 = s * NPW

        pltpu.sync_copy(src_hbm.at[w], src_v)
        pltpu.sync_copy(dst_hbm.at[w], dst_v)
        pltpu.sync_copy(zeros_hbm, zbuf)
        pltpu.sync_copy(zeros8_hbm, z8)
        pltpu.sync_copy(ones8_hbm, obuf)

        pltpu.sync_copy(z8, dacc_sh.at[pl.ds(row0, NPW)])
        plsc.subcore_barrier()

        @pl.loop(0, NCH)
        def _deg(k):
            pltpu.sync_copy(obuf, dacc_sh.at[dst_v.at[k]], add=True)

        plsc.subcore_barrier()
        pltpu.sync_copy(dacc_sh.at[pl.ds(row0, NPW)], z8)
        pltpu.sync_copy(z8, degp_hbm.at[c, pl.ds(row0, NPW)])

        @pl.loop(0, T24)
        def _t(t):
            @pl.loop(0, NWBC)
            def _z(i):
                pltpu.sync_copy(zbuf, acc_sh.at[pl.ds(row0 + i * _WB, _WB)])

            plsc.subcore_barrier()
            zt = z_hbm.at[t]

            pltpu.async_copy(zt.at[src_v.at[0]], gbuf, sem)

            @pl.loop(0, NCH - 2, step=2)
            def _k(g):
                dB = pltpu.async_copy(zt.at[src_v.at[g + 1]], gbuf2, sem2)
                pltpu.make_async_copy(zt.at[src_v.at[g]], gbuf, sem).wait()
                pltpu.sync_copy(gbuf, acc_sh.at[dst_v.at[g]], add=True)
                pltpu.async_copy(zt.at[src_v.at[g + 2]], gbuf, sem)
                dB.wait()
                pltpu.sync_copy(gbuf2, acc_sh.at[dst_v.at[g + 1]], add=True)

            dB = pltpu.async_copy(zt.at[src_v.at[NCH - 1]], gbuf2, sem2)
            pltpu.make_async_copy(zt.at[src_v.at[NCH - 2]], gbuf, sem).wait()
            pltpu.sync_copy(gbuf, acc_sh.at[dst_v.at[NCH - 2]], add=True)
            dB.wait()
            pltpu.sync_copy(gbuf2, acc_sh.at[dst_v.at[NCH - 1]], add=True)

            plsc.subcore_barrier()

            @pl.loop(0, NWBC)
            def _wbk(i):
                pltpu.sync_copy(acc_sh.at[pl.ds(row0 + i * _WB, _WB)], wb)
                pltpu.sync_copy(
                    wb, spart_hbm.at[c, t, pl.ds(row0 + i * _WB, _WB)])

    mesh = plsc.VectorSubcoreMesh(
        core_axis_name="c", subcore_axis_name="s",
        num_cores=_NC, num_subcores=_NS)
    return pl.kernel(
        body,
        out_type=[
            jax.ShapeDtypeStruct((_NC, T24, NPAD, H), jnp.float32),
            jax.ShapeDtypeStruct((_NC, NPAD, 8), jnp.float32),
        ],
        mesh=mesh,
        scratch_types=[
            pltpu.VMEM((NCH, _CH), jnp.int32),
            pltpu.VMEM((NCH, _CH), jnp.int32),
            pltpu.VMEM((_CH, H), jnp.float32),
            pltpu.VMEM((_CH, H), jnp.float32),
            pltpu.VMEM((_WB, H), jnp.float32),
            pltpu.VMEM((_WB, H), jnp.float32),
            pltpu.VMEM((_CH, 8), jnp.float32),
            pltpu.VMEM((NPW, 8), jnp.float32),
            pltpu.VMEM_SHARED((NPAD, H), jnp.float32),
            pltpu.VMEM_SHARED((NPAD, 8), jnp.float32),
            pltpu.SemaphoreType.DMA,
            pltpu.SemaphoreType.DMA,
        ],
        compiler_params=pltpu.CompilerParams(use_tc_tiling_on_sc=False),
    )


def _tail_body(B, T, H, r_ref, sp_ref, dg_ref, emb_ref, bl_ref, br_ref,
               wih_ref, whh_ref, bih_ref, bhh_ref, wfc_ref, bfc_ref, out_ref):
    d = dg_ref[0, :, 0:1] + dg_ref[1, :, 0:1]
    rinv = 1.0 / jnp.maximum(d, 1.0)
    emb = emb_ref[...]
    bl = bl_ref[...]
    br = br_ref[...]
    wih = wih_ref[...]
    whh = whh_ref[...]
    bih = bih_ref[...]
    bhh = bhh_ref[...]
    bn = emb.shape[0]
    h = jnp.zeros((B * bn, H), jnp.float32)
    for t in range(T):
        inps = []
        for b in range(B):
            bt = b * T + t
            sp = sp_ref[0, bt] + sp_ref[1, bt]
            out1 = jax.nn.relu(r_ref[bt] + br + sp * rinv + bl)
            inps.append(out1 + emb)
        inp = jnp.concatenate(inps, axis=0)
        gx = jnp.dot(inp, wih, preferred_element_type=jnp.float32) + bih
        gh = jnp.dot(h, whh, preferred_element_type=jnp.float32) + bhh
        r_g = jax.nn.sigmoid(gx[:, :H] + gh[:, :H])
        z_g = jax.nn.sigmoid(gx[:, H:2 * H] + gh[:, H:2 * H])
        n_g = jnp.tanh(gx[:, 2 * H:] + r_g * gh[:, 2 * H:])
        h = (1.0 - z_g) * n_g + z_g * h
    o = jnp.dot(h, wfc_ref[...], preferred_element_type=jnp.float32) + bfc_ref[...]
    for b in range(B):
        out_ref[b] = o[b * bn:(b + 1) * bn]


def _make_tail(B, N, T, H, TOUT, bn):
    return pl.pallas_call(
        functools.partial(_tail_body, B, T, H),
        grid=(N // bn,),
        in_specs=[
            pl.BlockSpec((B * T, bn, H), lambda j: (0, j, 0)),
            pl.BlockSpec((_NC, B * T, bn, H), lambda j: (0, 0, j, 0)),
            pl.BlockSpec((_NC, bn, 8), lambda j: (0, j, 0)),
            pl.BlockSpec((bn, H), lambda j: (j, 0)),
            pl.BlockSpec((1, H), lambda j: (0, 0)),
            pl.BlockSpec((1, H), lambda j: (0, 0)),
            pl.BlockSpec((H, 3 * H), lambda j: (0, 0)),
            pl.BlockSpec((H, 3 * H), lambda j: (0, 0)),
            pl.BlockSpec((1, 3 * H), lambda j: (0, 0)),
            pl.BlockSpec((1, 3 * H), lambda j: (0, 0)),
            pl.BlockSpec((H, TOUT), lambda j: (0, 0)),
            pl.BlockSpec((1, TOUT), lambda j: (0, 0)),
        ],
        out_specs=pl.BlockSpec((B, bn, TOUT), lambda j: (0, j, 0)),
        out_shape=jax.ShapeDtypeStruct((B, N, TOUT), jnp.float32),
    )


def kernel(X, edge_index, cent_n_id, Wl, bl, Wr, br, node_emb,
           W_ih, W_hh, b_ih, b_hh, W_fc, b_fc):
    B, N, T, F = X.shape
    H = Wl.shape[1]
    TOUT = W_fc.shape[1]
    E = edge_index.shape[1]
    T24 = B * T
    EPW = E // _NW
    NCH = EPW // _CH
    NPAD = -(-N // (_NS * _WB)) * (_NS * _WB)
    NPW = NPAD // _NS

    z, r = _make_proj(B, N, T, F, H, bn=1000)(X, Wl, Wr)

    src_r = edge_index[0].reshape(_NW, NCH, _CH)
    dst_r = edge_index[1].reshape(_NW, NCH, _CH)
    zeros = jnp.zeros((_WB, H), jnp.float32)
    zeros8 = jnp.zeros((NPW, 8), jnp.float32)
    ones8 = jnp.ones((_CH, 8), jnp.float32)
    spart, degp = _make_sc_segsum(T24, NPAD, H, NCH)(
        z, src_r, dst_r, zeros, zeros8, ones8)

    emb = node_emb[cent_n_id]
    out = _make_tail(B, N, T, H, TOUT, bn=400)(
        r, spart, degp, emb,
        bl.reshape(1, H), br.reshape(1, H), W_ih, W_hh,
        b_ih.reshape(1, 3 * H), b_hh.reshape(1, 3 * H),
        W_fc, b_fc.reshape(1, TOUT))
    return out


# speedup vs baseline: 15.0405x; 1.0099x over previous
import functools

import jax
import jax.numpy as jnp
from jax import lax
from jax.experimental import pallas as pl
from jax.experimental.pallas import tpu as pltpu
from jax.experimental.pallas import tpu_sc as plsc

_NC = 2
_NS = 16
_NW = _NC * _NS

_CH = 125
_WB = 160


def _proj_body(T, H, x_ref, wl_ref, wr_ref, z_ref, r_ref):
    x = x_ref[0]
    bn = x.shape[0]
    xf = x.reshape(bn * T, x.shape[2])
    z = jnp.dot(xf, wl_ref[...], preferred_element_type=jnp.float32)
    r = jnp.dot(xf, wr_ref[...], preferred_element_type=jnp.float32)
    z_ref[...] = z.reshape(bn, T, H).transpose(1, 0, 2)
    r_ref[...] = r.reshape(bn, T, H).transpose(1, 0, 2)


def _make_proj(B, N, T, F, H, bn):
    return pl.pallas_call(
        functools.partial(_proj_body, T, H),
        grid=(B, N // bn),
        in_specs=[
            pl.BlockSpec((1, bn, T, F), lambda b, j: (b, j, 0, 0)),
            pl.BlockSpec((F, H), lambda b, j: (0, 0)),
            pl.BlockSpec((F, H), lambda b, j: (0, 0)),
        ],
        out_specs=[
            pl.BlockSpec((T, bn, H), lambda b, j: (b, j, 0)),
            pl.BlockSpec((T, bn, H), lambda b, j: (b, j, 0)),
        ],
        out_shape=[
            jax.ShapeDtypeStruct((B * T, N, H), jnp.float32),
            jax.ShapeDtypeStruct((B * T, N, H), jnp.float32),
        ],
    )


def _make_sc_segsum(T24, NPAD, H, NCH):
    NPW = NPAD // _NS
    NWBC = NPW // _WB

    def body(z_hbm, src_hbm, dst_hbm, zeros_hbm, zeros8_hbm, ones8_hbm,
             spart_hbm, degp_hbm,
             src_v, dst_v, gbuf, gbuf2, zbuf, wb, obuf, z8, acc_sh, dacc_sh,
             sem, sem2):
        c = lax.axis_index("c")
        s = lax.axis_index("s")
        w = s * _NC + c
        row0 = s * NPW

        pltpu.sync_copy(src_hbm.at[w], src_v)
        pltpu.sync_copy(dst_hbm.at[w], dst_v)
        pltpu.sync_copy(zeros_hbm, zbuf)
        pltpu.sync_copy(zeros8_hbm, z8)
        pltpu.sync_copy(ones8_hbm, obuf)

        pltpu.sync_copy(z8, dacc_sh.at[pl.ds(row0, NPW)])
        plsc.subcore_barrier()

        @pl.loop(0, NCH)
        def _deg(k):
            pltpu.sync_copy(obuf, dacc_sh.at[dst_v.at[k]], add=True)

        plsc.subcore_barrier()
        pltpu.sync_copy(dacc_sh.at[pl.ds(row0, NPW)], z8)
        pltpu.sync_copy(z8, degp_hbm.at[c, pl.ds(row0, NPW)])

        @pl.loop(0, NWBC)
        def _z0(i):
            pltpu.sync_copy(zbuf, acc_sh.at[pl.ds(row0 + i * _WB, _WB)])

        plsc.subcore_barrier()
        pltpu.async_copy(z_hbm.at[0].at[src_v.at[0]], gbuf, sem)

        @pl.loop(0, T24)
        def _t(t):
            zt = z_hbm.at[t]

            @pl.loop(0, NCH - 2, step=2)
            def _k(g):
                dB = pltpu.async_copy(zt.at[src_v.at[g + 1]], gbuf2, sem2)
                pltpu.make_async_copy(zt.at[src_v.at[g]], gbuf, sem).wait()
                pltpu.sync_copy(gbuf, acc_sh.at[dst_v.at[g]], add=True)
                pltpu.async_copy(zt.at[src_v.at[g + 2]], gbuf, sem)
                dB.wait()
                pltpu.sync_copy(gbuf2, acc_sh.at[dst_v.at[g + 1]], add=True)

            dB = pltpu.async_copy(zt.at[src_v.at[NCH - 1]], gbuf2, sem2)
            pltpu.make_async_copy(zt.at[src_v.at[NCH - 2]], gbuf, sem).wait()
            pltpu.sync_copy(gbuf, acc_sh.at[dst_v.at[NCH - 2]], add=True)
            dB.wait()
            pltpu.sync_copy(gbuf2, acc_sh.at[dst_v.at[NCH - 1]], add=True)

            plsc.subcore_barrier()
            tn = jnp.minimum(t + 1, T24 - 1)
            pltpu.async_copy(z_hbm.at[tn].at[src_v.at[0]], gbuf, sem)

            @pl.loop(0, NWBC)
            def _wbk(i):
                pltpu.sync_copy(acc_sh.at[pl.ds(row0 + i * _WB, _WB)], wb)
                pltpu.sync_copy(
                    wb, spart_hbm.at[c, t, pl.ds(row0 + i * _WB, _WB)])
                pltpu.sync_copy(zbuf, acc_sh.at[pl.ds(row0 + i * _WB, _WB)])

            plsc.subcore_barrier()

        pltpu.make_async_copy(
            z_hbm.at[T24 - 1].at[src_v.at[0]], gbuf, sem).wait()

    mesh = plsc.VectorSubcoreMesh(
        core_axis_name="c", subcore_axis_name="s",
        num_cores=_NC, num_subcores=_NS)
    return pl.kernel(
        body,
        out_type=[
            jax.ShapeDtypeStruct((_NC, T24, NPAD, H), jnp.float32),
            jax.ShapeDtypeStruct((_NC, NPAD, 8), jnp.float32),
        ],
        mesh=mesh,
        scratch_types=[
            pltpu.VMEM((NCH, _CH), jnp.int32),
            pltpu.VMEM((NCH, _CH), jnp.int32),
            pltpu.VMEM((_CH, H), jnp.float32),
            pltpu.VMEM((_CH, H), jnp.float32),
            pltpu.VMEM((_WB, H), jnp.float32),
            pltpu.VMEM((_WB, H), jnp.float32),
            pltpu.VMEM((_CH, 8), jnp.float32),
            pltpu.VMEM((NPW, 8), jnp.float32),
            pltpu.VMEM_SHARED((NPAD, H), jnp.float32),
            pltpu.VMEM_SHARED((NPAD, 8), jnp.float32),
            pltpu.SemaphoreType.DMA,
            pltpu.SemaphoreType.DMA,
        ],
        compiler_params=pltpu.CompilerParams(use_tc_tiling_on_sc=False),
    )


def _tail_body(B, T, H, r_ref, sp_ref, dg_ref, emb_ref, bl_ref, br_ref,
               wih_ref, whh_ref, bih_ref, bhh_ref, wfc_ref, bfc_ref, out_ref):
    d = dg_ref[0, :, 0:1] + dg_ref[1, :, 0:1]
    rinv = 1.0 / jnp.maximum(d, 1.0)
    emb = emb_ref[...]
    bl = bl_ref[...]
    br = br_ref[...]
    wih = wih_ref[...]
    whh = whh_ref[...]
    bih = bih_ref[...]
    bhh = bhh_ref[...]
    bn = emb.shape[0]
    h = jnp.zeros((B * bn, H), jnp.float32)
    for t in range(T):
        inps = []
        for b in range(B):
            bt = b * T + t
            sp = sp_ref[0, bt] + sp_ref[1, bt]
            out1 = jax.nn.relu(r_ref[bt] + br + sp * rinv + bl)
            inps.append(out1 + emb)
        inp = jnp.concatenate(inps, axis=0)
        gx = jnp.dot(inp, wih, preferred_element_type=jnp.float32) + bih
        gh = jnp.dot(h, whh, preferred_element_type=jnp.float32) + bhh
        r_g = jax.nn.sigmoid(gx[:, :H] + gh[:, :H])
        z_g = jax.nn.sigmoid(gx[:, H:2 * H] + gh[:, H:2 * H])
        n_g = jnp.tanh(gx[:, 2 * H:] + r_g * gh[:, 2 * H:])
        h = (1.0 - z_g) * n_g + z_g * h
    o = jnp.dot(h, wfc_ref[...], preferred_element_type=jnp.float32) + bfc_ref[...]
    for b in range(B):
        out_ref[b] = o[b * bn:(b + 1) * bn]


def _make_tail(B, N, T, H, TOUT, bn):
    return pl.pallas_call(
        functools.partial(_tail_body, B, T, H),
        grid=(N // bn,),
        in_specs=[
            pl.BlockSpec((B * T, bn, H), lambda j: (0, j, 0)),
            pl.BlockSpec((_NC, B * T, bn, H), lambda j: (0, 0, j, 0)),
            pl.BlockSpec((_NC, bn, 8), lambda j: (0, j, 0)),
            pl.BlockSpec((bn, H), lambda j: (j, 0)),
            pl.BlockSpec((1, H), lambda j: (0, 0)),
            pl.BlockSpec((1, H), lambda j: (0, 0)),
            pl.BlockSpec((H, 3 * H), lambda j: (0, 0)),
            pl.BlockSpec((H, 3 * H), lambda j: (0, 0)),
            pl.BlockSpec((1, 3 * H), lambda j: (0, 0)),
            pl.BlockSpec((1, 3 * H), lambda j: (0, 0)),
            pl.BlockSpec((H, TOUT), lambda j: (0, 0)),
            pl.BlockSpec((1, TOUT), lambda j: (0, 0)),
        ],
        out_specs=pl.BlockSpec((B, bn, TOUT), lambda j: (0, j, 0)),
        out_shape=jax.ShapeDtypeStruct((B, N, TOUT), jnp.float32),
    )


def kernel(X, edge_index, cent_n_id, Wl, bl, Wr, br, node_emb,
           W_ih, W_hh, b_ih, b_hh, W_fc, b_fc):
    B, N, T, F = X.shape
    H = Wl.shape[1]
    TOUT = W_fc.shape[1]
    E = edge_index.shape[1]
    T24 = B * T
    EPW = E // _NW
    NCH = EPW // _CH
    NPAD = -(-N // (_NS * _WB)) * (_NS * _WB)
    NPW = NPAD // _NS

    z, r = _make_proj(B, N, T, F, H, bn=1000)(X, Wl, Wr)

    src_r = edge_index[0].reshape(_NW, NCH, _CH)
    dst_r = edge_index[1].reshape(_NW, NCH, _CH)
    zeros = jnp.zeros((_WB, H), jnp.float32)
    zeros8 = jnp.zeros((NPW, 8), jnp.float32)
    ones8 = jnp.ones((_CH, 8), jnp.float32)
    spart, degp = _make_sc_segsum(T24, NPAD, H, NCH)(
        z, src_r, dst_r, zeros, zeros8, ones8)

    emb = node_emb[cent_n_id]
    out = _make_tail(B, N, T, H, TOUT, bn=400)(
        r, spart, degp, emb,
        bl.reshape(1, H), br.reshape(1, H), W_ih, W_hh,
        b_ih.reshape(1, 3 * H), b_hh.reshape(1, 3 * H),
        W_fc, b_fc.reshape(1, TOUT))
    return out
